# baseline (speedup 1.0000x reference)
import functools
import jax
import jax.numpy as jnp
from jax import lax
from jax.experimental import pallas as pl
from jax.experimental.pallas import tpu as pltpu, tpu_sc as plsc

ROWS = 16384
DIM = 2048
L = 16
NC = 2
NS = 16
NW = NC * NS
ROWS_PER_W = ROWS // NW
CHUNK = 4
CB = CHUNK * DIM
N_CHUNKS = ROWS_PER_W // CHUNK
JBLKS = DIM // L
GROUP = 32

_mesh = plsc.VectorSubcoreMesh(core_axis_name="c", subcore_axis_name="s")


@functools.partial(
    pl.kernel,
    out_type=jax.ShapeDtypeStruct((ROWS, DIM), jnp.float32),
    mesh=_mesh,
    compiler_params=pltpu.CompilerParams(needs_layout_passes=False),
    scratch_types=[
        pltpu.VMEM((DIM,), jnp.int32),
        pltpu.VMEM((CB,), jnp.float32),
        pltpu.VMEM((CB,), jnp.float32),
        pltpu.VMEM((CB,), jnp.float32),
        pltpu.VMEM((CB,), jnp.float32),
        pltpu.SemaphoreType.DMA,
        pltpu.SemaphoreType.DMA,
        pltpu.SemaphoreType.DMA,
        pltpu.SemaphoreType.DMA,
    ],
)
def _permute_sc(x_hbm, idx_hbm, out_hbm, idx_v, in0, in1, out0, out1,
                in_s0, in_s1, out_s0, out_s1):
    wid = lax.axis_index("s") * NC + lax.axis_index("c")
    w_row = wid * ROWS_PER_W

    in_bufs = (in0, in1)
    out_bufs = (out0, out1)
    in_sems = (in_s0, in_s1)
    out_sems = (out_s0, out_s1)

    pltpu.sync_copy(idx_hbm, idx_v)

    def start_in(c, b):
        row = w_row + c * CHUNK
        for i in range(CHUNK):
            pltpu.async_copy(x_hbm.at[row + i],
                             in_bufs[b].at[pl.ds(i * DIM, DIM)], in_sems[b])

    def wait_in(b):
        for i in range(CHUNK):
            pltpu.make_async_copy(x_hbm.at[0],
                                  in_bufs[b].at[pl.ds(i * DIM, DIM)],
                                  in_sems[b]).wait()

    def start_out(c, b):
        row = w_row + c * CHUNK
        for i in range(CHUNK):
            pltpu.async_copy(out_bufs[b].at[pl.ds(i * DIM, DIM)],
                             out_hbm.at[row + i], out_sems[b])

    def wait_out(b):
        for i in range(CHUNK):
            pltpu.make_async_copy(out_bufs[b].at[pl.ds(i * DIM, DIM)],
                                  out_hbm.at[0], out_sems[b]).wait()

    def gather_chunk(b):
        in_buf = in_bufs[b]
        out_buf = out_bufs[b]
        for g in range(JBLKS // GROUP):
            gbase = g * (GROUP * L)
            idxs = [idx_v[pl.ds(gbase + j * L, L)] for j in range(GROUP)]

            @plsc.parallel_loop(0, CHUNK)
            def row_body(i):
                row_ref = in_buf.at[pl.ds(i * DIM, DIM)]
                obase = i * DIM + gbase
                for j in range(GROUP):
                    v = plsc.load_gather(row_ref, [idxs[j]])
                    out_buf[pl.ds(obase + j * L, L)] = v

    start_in(0, 0)
    start_in(1, 1)

    def pair_body(k, acc):
        for b in range(2):
            c = 2 * k + b
            wait_in(b)

            @pl.when(c >= 2)
            def _():
                wait_out(b)

            gather_chunk(b)

            @pl.when(c + 2 < N_CHUNKS)
            def _():
                start_in(c + 2, b)

            start_out(c, b)
        return acc

    lax.fori_loop(0, N_CHUNKS // 2, pair_body, 0)
    for b in range(2):
        wait_out(b)


def kernel(x, indices):
    return _permute_sc(x, indices)


# speedup vs baseline: 1.2442x; 1.2442x over previous
import functools
import jax
import jax.numpy as jnp
from jax import lax
from jax.experimental import pallas as pl
from jax.experimental.pallas import tpu as pltpu, tpu_sc as plsc

ROWS = 16384
DIM = 2048
L = 16
NC = 2
NS = 16
NW = NC * NS
ROWS_PER_W = ROWS // NW
CIN = 16
COUT = 8
N16 = ROWS_PER_W // CIN
JBLKS = DIM // L
GROUP = 32

_mesh = plsc.VectorSubcoreMesh(core_axis_name="c", subcore_axis_name="s")


@functools.partial(
    pl.kernel,
    out_type=jax.ShapeDtypeStruct((ROWS, DIM), jnp.float32),
    mesh=_mesh,
    compiler_params=pltpu.CompilerParams(needs_layout_passes=False),
    scratch_types=[
        pltpu.VMEM((DIM,), jnp.int32),
        pltpu.VMEM((CIN * DIM,), jnp.float32),
        pltpu.VMEM((CIN * DIM,), jnp.float32),
        pltpu.VMEM((COUT * DIM,), jnp.float32),
        pltpu.VMEM((COUT * DIM,), jnp.float32),
        pltpu.SemaphoreType.DMA,
        pltpu.SemaphoreType.DMA,
        pltpu.SemaphoreType.DMA,
        pltpu.SemaphoreType.DMA,
    ],
)
def _permute_sc(x_hbm, idx_hbm, out_hbm, idx_v, in0, in1, out0, out1,
                in_s0, in_s1, out_s0, out_s1):
    wid = lax.axis_index("s") * NC + lax.axis_index("c")
    w_row = wid * ROWS_PER_W

    in_bufs = (in0, in1)
    out_bufs = (out0, out1)
    in_sems = (in_s0, in_s1)
    out_sems = (out_s0, out_s1)

    pltpu.sync_copy(idx_hbm, idx_v)

    def start_in(c16, b):
        row = w_row + c16 * CIN
        for i in range(CIN):
            pltpu.async_copy(x_hbm.at[row + i],
                             in_bufs[b].at[pl.ds(i * DIM, DIM)], in_sems[b])

    def wait_in(b):
        for i in range(CIN):
            pltpu.make_async_copy(x_hbm.at[0],
                                  in_bufs[b].at[pl.ds(i * DIM, DIM)],
                                  in_sems[b]).wait()

    def start_out(c8, h):
        row = w_row + c8 * COUT
        for i in range(COUT):
            pltpu.async_copy(out_bufs[h].at[pl.ds(i * DIM, DIM)],
                             out_hbm.at[row + i], out_sems[h])

    def wait_out(h):
        for i in range(COUT):
            pltpu.make_async_copy(out_bufs[h].at[pl.ds(i * DIM, DIM)],
                                  out_hbm.at[0], out_sems[h]).wait()

    def gather_half(b, h):
        in_buf = in_bufs[b]
        out_buf = out_bufs[h]
        for g in range(JBLKS // GROUP):
            gbase = g * (GROUP * L)
            idxs = [idx_v[pl.ds(gbase + j * L, L)] for j in range(GROUP)]

            @plsc.parallel_loop(0, COUT)
            def row_body(i):
                row_ref = in_buf.at[pl.ds((h * COUT + i) * DIM, DIM)]
                obase = i * DIM + gbase
                for j in range(GROUP):
                    v = plsc.load_gather(row_ref, [idxs[j]])
                    out_buf[pl.ds(obase + j * L, L)] = v

    start_in(0, 0)
    start_in(1, 1)

    def pair_body(k, acc):
        for b in range(2):
            c16 = 2 * k + b
            wait_in(b)
            for h in range(2):
                @pl.when(c16 >= 1)
                def _():
                    wait_out(h)

                gather_half(b, h)
                start_out(c16 * 2 + h, h)

            @pl.when(c16 + 2 < N16)
            def _():
                start_in(c16 + 2, b)
        return acc

    lax.fori_loop(0, N16 // 2, pair_body, 0)
    for h in range(2):
        wait_out(h)


def kernel(x, indices):
    return _permute_sc(x, indices)


# speedup vs baseline: 1.2453x; 1.0009x over previous
import functools
import jax
import jax.numpy as jnp
from jax import lax
from jax.experimental import pallas as pl
from jax.experimental.pallas import tpu as pltpu, tpu_sc as plsc

ROWS = 16384
DIM = 2048
L = 16
NC = 2
NS = 16
NW = NC * NS
ROWS_PER_W = ROWS // NW
CHUNK = 8
CB = CHUNK * DIM
N_CHUNKS = ROWS_PER_W // CHUNK
JBLKS = DIM // L
GROUP = 32

_mesh = plsc.VectorSubcoreMesh(core_axis_name="c", subcore_axis_name="s")


@functools.partial(
    pl.kernel,
    out_type=jax.ShapeDtypeStruct((ROWS, DIM), jnp.float32),
    mesh=_mesh,
    compiler_params=pltpu.CompilerParams(needs_layout_passes=False),
    scratch_types=[
        pltpu.VMEM((DIM,), jnp.int32),
        pltpu.VMEM((CB,), jnp.float32),
        pltpu.VMEM((CB,), jnp.float32),
        pltpu.VMEM((CB,), jnp.float32),
        pltpu.VMEM((CB,), jnp.float32),
        pltpu.VMEM((CB,), jnp.float32),
        pltpu.VMEM((CB,), jnp.float32),
        pltpu.SemaphoreType.DMA,
        pltpu.SemaphoreType.DMA,
        pltpu.SemaphoreType.DMA,
        pltpu.SemaphoreType.DMA,
        pltpu.SemaphoreType.DMA,
        pltpu.SemaphoreType.DMA,
    ],
)
def _permute_sc(x_hbm, idx_hbm, out_hbm, idx_v, in0, in1, in2, in3,
                out0, out1, in_s0, in_s1, in_s2, in_s3, out_s0, out_s1):
    wid = lax.axis_index("s") * NC + lax.axis_index("c")
    w_row = wid * ROWS_PER_W

    in_bufs = (in0, in1, in2, in3)
    out_bufs = (out0, out1)
    in_sems = (in_s0, in_s1, in_s2, in_s3)
    out_sems = (out_s0, out_s1)

    pltpu.sync_copy(idx_hbm, idx_v)

    def start_in(c, b):
        row = w_row + c * CHUNK
        for i in range(CHUNK):
            pltpu.async_copy(x_hbm.at[row + i],
                             in_bufs[b].at[pl.ds(i * DIM, DIM)], in_sems[b])

    def wait_in(b):
        for i in range(CHUNK):
            pltpu.make_async_copy(x_hbm.at[0],
                                  in_bufs[b].at[pl.ds(i * DIM, DIM)],
                                  in_sems[b]).wait()

    def start_out(c, b):
        row = w_row + c * CHUNK
        for i in range(CHUNK):
            pltpu.async_copy(out_bufs[b].at[pl.ds(i * DIM, DIM)],
                             out_hbm.at[row + i], out_sems[b])

    def wait_out(b):
        for i in range(CHUNK):
            pltpu.make_async_copy(out_bufs[b].at[pl.ds(i * DIM, DIM)],
                                  out_hbm.at[0], out_sems[b]).wait()

    def gather_chunk(b):
        in_buf = in_bufs[b]
        out_buf = out_bufs[b % 2]
        for g in range(JBLKS // GROUP):
            gbase = g * (GROUP * L)
            idxs = [idx_v[pl.ds(gbase + j * L, L)] for j in range(GROUP)]

            @plsc.parallel_loop(0, CHUNK)
            def row_body(i):
                row_ref = in_buf.at[pl.ds(i * DIM, DIM)]
                obase = i * DIM + gbase
                for j in range(GROUP):
                    v = plsc.load_gather(row_ref, [idxs[j]])
                    out_buf[pl.ds(obase + j * L, L)] = v

    for b in range(4):
        start_in(b, b)

    def quad_body(k, acc):
        for b in range(4):
            c = 4 * k + b
            wait_in(b)

            @pl.when(c >= 2)
            def _():
                wait_out(b % 2)

            gather_chunk(b)

            @pl.when(c + 4 < N_CHUNKS)
            def _():
                start_in(c + 4, b)

            start_out(c, b % 2)
        return acc

    lax.fori_loop(0, N_CHUNKS // 4, quad_body, 0)
    for b in range(2):
        wait_out(b)


def kernel(x, indices):
    return _permute_sc(x, indices)


# speedup vs baseline: 1.3837x; 1.1111x over previous
import functools
import jax
import jax.numpy as jnp
from jax import lax
from jax.experimental import pallas as pl
from jax.experimental.pallas import tpu as pltpu, tpu_sc as plsc

ROWS = 16384
DIM = 2048
L = 16
NC = 2
NS = 16
NW = NC * NS
ROWS_PER_W = ROWS // NW
CHUNK = 8
CB = CHUNK * DIM
N_CHUNKS = ROWS_PER_W // CHUNK
JBLKS = DIM // L
GROUP = 32

_mesh = plsc.VectorSubcoreMesh(core_axis_name="c", subcore_axis_name="s")


@functools.partial(
    pl.kernel,
    out_type=jax.ShapeDtypeStruct((ROWS, DIM), jnp.float32),
    mesh=_mesh,
    compiler_params=pltpu.CompilerParams(needs_layout_passes=False),
    scratch_types=[
        pltpu.VMEM((DIM,), jnp.int32),
        pltpu.VMEM((CB,), jnp.float32),
        pltpu.VMEM((CB,), jnp.float32),
        pltpu.VMEM((CB,), jnp.float32),
        pltpu.VMEM((CB,), jnp.float32),
        pltpu.SemaphoreType.DMA,
        pltpu.SemaphoreType.DMA,
        pltpu.SemaphoreType.DMA,
        pltpu.SemaphoreType.DMA,
    ],
)
def _permute_sc(x_hbm, idx_hbm, out_hbm, idx_v, in0, in1, out0, out1,
                in_s0, in_s1, out_s0, out_s1):
    wid = lax.axis_index("s") * NC + lax.axis_index("c")
    w_row = wid * ROWS_PER_W

    in_bufs = (in0, in1)
    out_bufs = (out0, out1)
    in_sems = (in_s0, in_s1)
    out_sems = (out_s0, out_s1)

    pltpu.sync_copy(idx_hbm, idx_v)

    def start_in(c, b):
        row = w_row + c * CHUNK
        for i in range(CHUNK):
            pltpu.async_copy(x_hbm.at[row + i],
                             in_bufs[b].at[pl.ds(i * DIM, DIM)], in_sems[b])

    def wait_in(b):
        for i in range(CHUNK):
            pltpu.make_async_copy(x_hbm.at[0],
                                  in_bufs[b].at[pl.ds(i * DIM, DIM)],
                                  in_sems[b]).wait()

    def start_out(c, b):
        row = w_row + c * CHUNK
        for i in range(CHUNK):
            pltpu.async_copy(out_bufs[b].at[pl.ds(i * DIM, DIM)],
                             out_hbm.at[row + i], out_sems[b])

    def wait_out(b):
        for i in range(CHUNK):
            pltpu.make_async_copy(out_bufs[b].at[pl.ds(i * DIM, DIM)],
                                  out_hbm.at[0], out_sems[b]).wait()

    def gather_chunk(b):
        in_buf = in_bufs[b]
        out_buf = out_bufs[b]
        for g in range(JBLKS // GROUP):
            gbase = g * (GROUP * L)
            idxs = [idx_v[pl.ds(gbase + j * L, L)] for j in range(GROUP)]

            @plsc.parallel_loop(0, CHUNK)
            def row_body(i):
                row_ref = in_buf.at[pl.ds(i * DIM, DIM)]
                obase = i * DIM + gbase
                for j in range(GROUP):
                    v = plsc.load_gather(row_ref, [idxs[j]])
                    out_buf[pl.ds(obase + j * L, L)] = v

    start_in(0, 0)
    start_in(1, 1)

    def pair_body(k, acc):
        for b in range(2):
            c = 2 * k + b
            wait_in(b)

            @pl.when(c >= 2)
            def _():
                wait_out(b)

            gather_chunk(b)
            start_out(c, b)

            @pl.when(c + 2 < N_CHUNKS)
            def _():
                start_in(c + 2, b)
        return acc

    lax.fori_loop(0, N_CHUNKS // 2, pair_body, 0)
    for b in range(2):
        wait_out(b)


def kernel(x, indices):
    return _permute_sc(x, indices)
